# baseline (speedup 1.0000x reference)
import functools

import jax
import jax.numpy as jnp
from jax import lax
from jax.experimental import pallas as pl
from jax.experimental.pallas import tpu as pltpu
from jax.experimental.pallas import tpu_sc as plsc

_VOCAB = 1000000
_BATCH = 16384
_DIM = 32

_NC = 2
_NS = 16
_NW = _NC * _NS
_ASC = 12288
_ATC = _BATCH - _ASC
_BPW = _ASC // _NW
_CHUNK = 16


def _sc_part(idx_sc, table_t):
    mesh = plsc.VectorSubcoreMesh(core_axis_name="c", subcore_axis_name="s")

    @functools.partial(
        pl.kernel,
        mesh=mesh,
        out_type=jax.ShapeDtypeStruct((_DIM, _ASC), jnp.float32),
        scratch_types=[
            pltpu.VMEM((_BPW,), jnp.int32),
            pltpu.VMEM((_CHUNK, _DIM, 128), jnp.float32),
            pltpu.VMEM((_DIM, _BPW), jnp.float32),
            pltpu.SemaphoreType.DMA,
            pltpu.SemaphoreType.DMA,
        ],
        compiler_params=pltpu.CompilerParams(needs_layout_passes=False),
    )
    def gather_kernel(idx_hbm, table_hbm, out_hbm, idx_s, ring_v, block_v,
                      sem_i, sem_g):
        wid = lax.axis_index("s") * _NC + lax.axis_index("c")
        base = wid * _BPW
        pltpu.sync_copy(idx_hbm.at[pl.ds(base, _BPW)], idx_s)

        rows_lo = lax.iota(jnp.int32, 16)
        rows_hi = rows_lo + 16

        def chunk_body(ck, carry):
            iv = idx_s[pl.ds(ck * _CHUNK, _CHUNK)]
            for j in range(_CHUNK):
                col0 = pl.multiple_of(
                    lax.shift_right_logical(iv[j], 7) * 128, 128
                )
                pltpu.make_async_copy(
                    table_hbm.at[:, pl.ds(col0, 128)],
                    ring_v.at[j],
                    sem_g,
                ).start()
            for j in range(_CHUNK):
                pltpu.make_async_copy(
                    table_hbm.at[:, pl.ds(0, 128)], ring_v.at[j], sem_g
                ).wait()
            for j in range(_CHUNK):
                col = jnp.full((16,), iv[j] & 127, dtype=jnp.int32)
                pos = jnp.full((16,), ck * _CHUNK + j, dtype=jnp.int32)
                v_lo = plsc.load_gather(ring_v.at[j], [rows_lo, col])
                v_hi = plsc.load_gather(ring_v.at[j], [rows_hi, col])
                plsc.store_scatter(block_v, [rows_lo, pos], v_lo)
                plsc.store_scatter(block_v, [rows_hi, pos], v_hi)
            return carry

        lax.fori_loop(0, _BPW // _CHUNK, chunk_body, 0)

        pltpu.sync_copy(block_v, out_hbm.at[:, pl.ds(base, _BPW)])

    return gather_kernel(idx_sc, table_t)


def _tc_body(idx_ref, table_ref, out_ref, ring_v, sem0, sem1):
    i = pl.program_id(0)
    n_sub = 128 // _CHUNK
    sems = (sem0, sem1)

    def fire(sub, slot):
        for j in range(_CHUNK):
            r = idx_ref[i * 128 + sub * _CHUNK + j]
            col0 = pl.multiple_of(lax.shift_right_logical(r, 7) * 128, 128)
            pltpu.make_async_copy(
                table_ref.at[:, pl.ds(col0, 128)], ring_v.at[slot, j],
                sems[slot],
            ).start()

    def drain_and_extract(sub, slot):
        for j in range(_CHUNK):
            pltpu.make_async_copy(
                table_ref.at[:, pl.ds(0, 128)], ring_v.at[slot, j],
                sems[slot],
            ).wait()
        for j in range(_CHUNK):
            r = idx_ref[i * 128 + sub * _CHUNK + j]
            shift = (128 - (r & 127)) & 127
            rolled = pltpu.roll(ring_v[slot, j], shift, 1)
            p = sub * _CHUNK + j
            out_ref[:, p : p + 1] = rolled[:, 0:1]

    fire(0, 0)
    for sub in range(n_sub - 1):
        fire(sub + 1, (sub + 1) % 2)
        drain_and_extract(sub, sub % 2)
    drain_and_extract(n_sub - 1, (n_sub - 1) % 2)


def _tc_part(idx_tc, table_t):
    return pl.pallas_call(
        _tc_body,
        out_shape=jax.ShapeDtypeStruct((_DIM, _ATC), jnp.float32),
        grid=(_ATC // 128,),
        in_specs=[
            pl.BlockSpec(memory_space=pltpu.SMEM),
            pl.BlockSpec(memory_space=pltpu.HBM),
        ],
        out_specs=pl.BlockSpec((_DIM, 128), lambda i: (0, i)),
        scratch_shapes=[
            pltpu.VMEM((2, _CHUNK, _DIM, 128), jnp.float32),
            pltpu.SemaphoreType.DMA,
            pltpu.SemaphoreType.DMA,
        ],
    )(idx_tc, table_t)


@jax.jit
def kernel(indices, in_vec):
    idx32 = indices.astype(jnp.int32)
    table_t = in_vec.T
    out_tc = _tc_part(idx32[_ASC:], table_t)
    out_sc = _sc_part(idx32[:_ASC], table_t)
    return jnp.concatenate([out_sc, out_tc], axis=1).T


# speedup vs baseline: 1.0051x; 1.0051x over previous
import functools

import jax
import jax.numpy as jnp
from jax import lax
from jax.experimental import pallas as pl
from jax.experimental.pallas import tpu as pltpu
from jax.experimental.pallas import tpu_sc as plsc

_VOCAB = 1000000
_BATCH = 16384
_DIM = 32

_NC = 2
_NS = 16
_NW = _NC * _NS
_ASC = 12288
_ATC = _BATCH - _ASC
_BPW = _ASC // _NW
_CHUNK = 16


def _sc_part(idx_sc, table_t):
    mesh = plsc.VectorSubcoreMesh(core_axis_name="c", subcore_axis_name="s")

    @functools.partial(
        pl.kernel,
        mesh=mesh,
        out_type=jax.ShapeDtypeStruct((_DIM, _ASC), jnp.float32),
        scratch_types=[
            pltpu.VMEM((_BPW,), jnp.int32),
            pltpu.VMEM((_CHUNK, _DIM, 128), jnp.float32),
            pltpu.VMEM((_DIM, _BPW), jnp.float32),
            pltpu.SemaphoreType.DMA,
            pltpu.SemaphoreType.DMA,
        ],
        compiler_params=pltpu.CompilerParams(needs_layout_passes=False),
        cost_estimate=pl.CostEstimate(
            flops=1_000_000, transcendentals=0, bytes_accessed=200_000_000
        ),
    )
    def gather_kernel(idx_hbm, table_hbm, out_hbm, idx_s, ring_v, block_v,
                      sem_i, sem_g):
        wid = lax.axis_index("s") * _NC + lax.axis_index("c")
        base = wid * _BPW
        pltpu.sync_copy(idx_hbm.at[pl.ds(base, _BPW)], idx_s)

        rows_lo = lax.iota(jnp.int32, 16)
        rows_hi = rows_lo + 16

        def chunk_body(ck, carry):
            iv = idx_s[pl.ds(ck * _CHUNK, _CHUNK)]
            for j in range(_CHUNK):
                col0 = pl.multiple_of(
                    lax.shift_right_logical(iv[j], 7) * 128, 128
                )
                pltpu.make_async_copy(
                    table_hbm.at[:, pl.ds(col0, 128)],
                    ring_v.at[j],
                    sem_g,
                ).start()
            for j in range(_CHUNK):
                pltpu.make_async_copy(
                    table_hbm.at[:, pl.ds(0, 128)], ring_v.at[j], sem_g
                ).wait()
            for j in range(_CHUNK):
                col = jnp.full((16,), iv[j] & 127, dtype=jnp.int32)
                pos = jnp.full((16,), ck * _CHUNK + j, dtype=jnp.int32)
                v_lo = plsc.load_gather(ring_v.at[j], [rows_lo, col])
                v_hi = plsc.load_gather(ring_v.at[j], [rows_hi, col])
                plsc.store_scatter(block_v, [rows_lo, pos], v_lo)
                plsc.store_scatter(block_v, [rows_hi, pos], v_hi)
            return carry

        lax.fori_loop(0, _BPW // _CHUNK, chunk_body, 0)

        pltpu.sync_copy(block_v, out_hbm.at[:, pl.ds(base, _BPW)])

    return gather_kernel(idx_sc, table_t)


def _tc_body(idx_ref, table_ref, out_ref, ring_v, sem0, sem1):
    i = pl.program_id(0)
    n_sub = 128 // _CHUNK
    sems = (sem0, sem1)

    def fire(sub, slot):
        for j in range(_CHUNK):
            r = idx_ref[i * 128 + sub * _CHUNK + j]
            col0 = pl.multiple_of(lax.shift_right_logical(r, 7) * 128, 128)
            pltpu.make_async_copy(
                table_ref.at[:, pl.ds(col0, 128)], ring_v.at[slot, j],
                sems[slot],
            ).start()

    def drain_and_extract(sub, slot):
        for j in range(_CHUNK):
            pltpu.make_async_copy(
                table_ref.at[:, pl.ds(0, 128)], ring_v.at[slot, j],
                sems[slot],
            ).wait()
        for j in range(_CHUNK):
            r = idx_ref[i * 128 + sub * _CHUNK + j]
            shift = (128 - (r & 127)) & 127
            rolled = pltpu.roll(ring_v[slot, j], shift, 1)
            p = sub * _CHUNK + j
            out_ref[:, p : p + 1] = rolled[:, 0:1]

    fire(0, 0)
    for sub in range(n_sub - 1):
        fire(sub + 1, (sub + 1) % 2)
        drain_and_extract(sub, sub % 2)
    drain_and_extract(n_sub - 1, (n_sub - 1) % 2)


def _tc_part(idx_tc, table_t):
    return pl.pallas_call(
        _tc_body,
        out_shape=jax.ShapeDtypeStruct((_DIM, _ATC), jnp.float32),
        grid=(_ATC // 128,),
        in_specs=[
            pl.BlockSpec(memory_space=pltpu.SMEM),
            pl.BlockSpec(memory_space=pltpu.HBM),
        ],
        out_specs=pl.BlockSpec((_DIM, 128), lambda i: (0, i)),
        scratch_shapes=[
            pltpu.VMEM((2, _CHUNK, _DIM, 128), jnp.float32),
            pltpu.SemaphoreType.DMA,
            pltpu.SemaphoreType.DMA,
        ],
        cost_estimate=pl.CostEstimate(
            flops=1_000_000, transcendentals=0, bytes_accessed=70_000_000
        ),
    )(idx_tc, table_t)


@jax.jit
def kernel(indices, in_vec):
    idx32 = indices.astype(jnp.int32)
    table_t = in_vec.T
    out_tc = _tc_part(idx32[_ASC:], table_t)
    out_sc = _sc_part(idx32[:_ASC], table_t)
    return jnp.concatenate([out_sc, out_tc], axis=1).T


# speedup vs baseline: 1.4031x; 1.3961x over previous
import functools

import jax
import jax.numpy as jnp
from jax import lax
from jax.experimental import pallas as pl
from jax.experimental.pallas import tpu as pltpu
from jax.experimental.pallas import tpu_sc as plsc

_VOCAB = 1000000
_BATCH = 16384
_DIM = 32

_NC = 2
_NS = 16
_NW = _NC * _NS
_BPW = _BATCH // _NW
_CHUNK = 16


@jax.jit
def kernel(indices, in_vec):
    mesh = plsc.VectorSubcoreMesh(core_axis_name="c", subcore_axis_name="s")

    @functools.partial(
        pl.kernel,
        mesh=mesh,
        out_type=jax.ShapeDtypeStruct((_DIM, _BATCH), jnp.float32),
        scratch_types=[
            pltpu.VMEM((_BPW,), jnp.int32),
            pltpu.VMEM((_CHUNK, _DIM, 128), jnp.float32),
            pltpu.VMEM((_DIM, _BPW), jnp.float32),
            pltpu.SemaphoreType.DMA,
            pltpu.SemaphoreType.DMA,
        ],
        compiler_params=pltpu.CompilerParams(needs_layout_passes=False),
    )
    def gather_kernel(idx_hbm, table_hbm, out_hbm, idx_s, ring_v, block_v,
                      sem_i, sem_g):
        wid = lax.axis_index("s") * _NC + lax.axis_index("c")
        base = wid * _BPW
        pltpu.sync_copy(idx_hbm.at[pl.ds(base, _BPW)], idx_s)

        rows_lo = lax.iota(jnp.int32, 16)
        rows_hi = rows_lo + 16

        def chunk_body(ck, carry):
            iv = idx_s[pl.ds(ck * _CHUNK, _CHUNK)]
            for j in range(_CHUNK):
                col0 = pl.multiple_of(
                    lax.shift_right_logical(iv[j], 7) * 128, 128
                )
                pltpu.make_async_copy(
                    table_hbm.at[:, pl.ds(col0, 128)],
                    ring_v.at[j],
                    sem_g,
                ).start()
            for j in range(_CHUNK):
                pltpu.make_async_copy(
                    table_hbm.at[:, pl.ds(0, 128)], ring_v.at[j], sem_g
                ).wait()
            for j in range(_CHUNK):
                col = jnp.full((16,), iv[j] & 127, dtype=jnp.int32)
                pos = jnp.full((16,), ck * _CHUNK + j, dtype=jnp.int32)
                v_lo = plsc.load_gather(ring_v.at[j], [rows_lo, col])
                v_hi = plsc.load_gather(ring_v.at[j], [rows_hi, col])
                plsc.store_scatter(block_v, [rows_lo, pos], v_lo)
                plsc.store_scatter(block_v, [rows_hi, pos], v_hi)
            return carry

        lax.fori_loop(0, _BPW // _CHUNK, chunk_body, 0)

        pltpu.sync_copy(block_v, out_hbm.at[:, pl.ds(base, _BPW)])

    out_t = gather_kernel(indices.astype(jnp.int32), in_vec.T)
    return out_t.T


# speedup vs baseline: 1.5019x; 1.0704x over previous
import functools

import jax
import jax.numpy as jnp
from jax import lax
from jax.experimental import pallas as pl
from jax.experimental.pallas import tpu as pltpu
from jax.experimental.pallas import tpu_sc as plsc

_VOCAB = 1000000
_BATCH = 16384
_DIM = 32

_NC = 2
_NS = 16
_NW = _NC * _NS
_BPW = _BATCH // _NW
_HCHUNK = 8


@jax.jit
def kernel(indices, in_vec):
    mesh = plsc.VectorSubcoreMesh(core_axis_name="c", subcore_axis_name="s")

    @functools.partial(
        pl.kernel,
        mesh=mesh,
        out_type=jax.ShapeDtypeStruct((_DIM, _BATCH), jnp.float32),
        scratch_types=[
            pltpu.VMEM((_BPW,), jnp.int32),
            pltpu.VMEM((2, _HCHUNK, _DIM, 128), jnp.float32),
            pltpu.VMEM((_DIM, _BPW), jnp.float32),
            pltpu.SemaphoreType.DMA,
            pltpu.SemaphoreType.DMA,
        ],
        compiler_params=pltpu.CompilerParams(needs_layout_passes=False),
    )
    def gather_kernel(idx_hbm, table_hbm, out_hbm, idx_s, ring_v, block_v,
                      sem_a, sem_b):
        wid = lax.axis_index("s") * _NC + lax.axis_index("c")
        base = wid * _BPW
        pltpu.sync_copy(idx_hbm.at[pl.ds(base, _BPW)], idx_s)

        rows_lo = lax.iota(jnp.int32, 16)
        rows_hi = rows_lo + 16
        sems = (sem_a, sem_b)

        def fire(slot, iv8):
            for j in range(_HCHUNK):
                col0 = pl.multiple_of(
                    lax.shift_right_logical(iv8[j], 7) * 128, 128
                )
                pltpu.make_async_copy(
                    table_hbm.at[:, pl.ds(col0, 128)],
                    ring_v.at[slot, j],
                    sems[slot],
                ).start()

        def drain_extract(slot, h, iv8):
            for j in range(_HCHUNK):
                pltpu.make_async_copy(
                    table_hbm.at[:, pl.ds(0, 128)],
                    ring_v.at[slot, j],
                    sems[slot],
                ).wait()
            for j in range(_HCHUNK):
                col = jnp.full((16,), iv8[j] & 127, dtype=jnp.int32)
                pos = jnp.full((16,), h * _HCHUNK + j, dtype=jnp.int32)
                v_lo = plsc.load_gather(ring_v.at[slot, j], [rows_lo, col])
                v_hi = plsc.load_gather(ring_v.at[slot, j], [rows_hi, col])
                plsc.store_scatter(block_v, [rows_lo, pos], v_lo)
                plsc.store_scatter(block_v, [rows_hi, pos], v_hi)

        n_pairs = _BPW // (2 * _HCHUNK)
        iv0 = idx_s[pl.ds(0, 16)]
        fire(0, iv0[0:8])
        fire(1, iv0[8:16])

        def pair_body(g, iv_cur):
            iv_next = idx_s[pl.ds((g + 1) * 16, 16)]
            drain_extract(0, 2 * g, iv_cur[0:8])
            fire(0, iv_next[0:8])
            drain_extract(1, 2 * g + 1, iv_cur[8:16])
            fire(1, iv_next[8:16])
            return iv_next

        iv_last = lax.fori_loop(0, n_pairs - 1, pair_body, iv0)
        drain_extract(0, 2 * (n_pairs - 1), iv_last[0:8])
        drain_extract(1, 2 * (n_pairs - 1) + 1, iv_last[8:16])

        pltpu.sync_copy(block_v, out_hbm.at[:, pl.ds(base, _BPW)])

    out_t = gather_kernel(indices.astype(jnp.int32), in_vec.T)
    return out_t.T
